# baseline (speedup 1.0000x reference)
import functools

import jax
import jax.numpy as jnp
from jax import lax
from jax.experimental import pallas as pl
from jax.experimental.pallas import tpu as pltpu
from jax.experimental.pallas import tpu_sc as plsc

B = 4
P = 8192
HW = 9600
M = 4096
GEO = 64
RGB = 512
FUS = 256

_HW_TILE = 1920
_GEO_ROWS = 4096

_NC = 2
_NS = 16
_NW = _NC * _NS
_RPW = (B * M) // _NW
_CH = 64
_NCH = _RPW // _CH
_WPB = M // _RPW
_LOG2P = 13


def _round_bf16_bits(x):
    b = lax.bitcast_convert_type(x, jnp.uint32)
    return (b + jnp.uint32(0x7FFF) + ((b >> 16) & jnp.uint32(1))) >> 16


def _pack_halves(r):
    e = _round_bf16_bits(r[:, :FUS // 2])
    o = _round_bf16_bits(r[:, FUS // 2:])
    return lax.bitcast_convert_type(e | (o << 16), jnp.int32)


def _rgb_proj_body(rgb_ref, wt_ref, out_ref):
    out_ref[...] = _pack_halves(lax.dot_general(
        rgb_ref[0], wt_ref[...],
        dimension_numbers=(((0,), (0,)), ((), ())),
        preferred_element_type=jnp.float32))


def _rgb_proj(rgb_f, wt_rgb):
    nt = HW // _HW_TILE
    return pl.pallas_call(
        _rgb_proj_body,
        grid=(B, nt),
        in_specs=[
            pl.BlockSpec((1, RGB, _HW_TILE), lambda b, t: (b, 0, t)),
            pl.BlockSpec((RGB, FUS), lambda b, t: (0, 0)),
        ],
        out_specs=pl.BlockSpec((_HW_TILE, FUS // 2), lambda b, t: (b * nt + t, 0)),
        out_shape=jax.ShapeDtypeStruct((B * HW, FUS // 2), jnp.int32),
    )(rgb_f, wt_rgb)


def _geo_proj_body(geo_hbm, wt_ref, b_ref, out_ref, buf, sem):
    r = pl.program_id(0)
    cp = pltpu.make_async_copy(
        geo_hbm.at[pl.ds(r * _GEO_ROWS, _GEO_ROWS)], buf, sem)
    cp.start()
    cp.wait()
    out_ref[...] = _pack_halves(lax.dot_general(
        buf[...], wt_ref[...],
        dimension_numbers=(((1,), (0,)), ((), ())),
        preferred_element_type=jnp.float32) + b_ref[...])


def _geo_proj(geo, wt_geo, bias):
    return pl.pallas_call(
        _geo_proj_body,
        grid=((B * P) // _GEO_ROWS,),
        in_specs=[
            pl.BlockSpec(memory_space=pl.ANY),
            pl.BlockSpec((GEO, FUS), lambda r: (0, 0)),
            pl.BlockSpec((1, FUS), lambda r: (0, 0)),
        ],
        out_specs=pl.BlockSpec((_GEO_ROWS, FUS // 2), lambda r: (r, 0)),
        out_shape=jax.ShapeDtypeStruct((B * P, FUS // 2), jnp.int32),
        scratch_shapes=[
            pltpu.VMEM((_GEO_ROWS, GEO), jnp.float32),
            pltpu.SemaphoreType.DMA,
        ],
    )(geo, wt_geo, bias)


def _sc_side_body(prgb, pgeo, idx_hbm, sel_m, sel_nm, out_m, out_nm,
                  idx_v, sel_v, rgbi_v, geoi_v,
                  rgb_b0, rgb_b1, rgb_b2, geo_b0, geo_b1, geo_b2,
                  st_b0, st_b1,
                  sga0, sga1, sga2, sgb0, sgb1, sgb2, so0, so1):
    wid = lax.axis_index("s") * _NC + lax.axis_index("c")
    batch = wid // _WPB
    base = wid * _RPW
    rgb_bufs = (rgb_b0, rgb_b1, rgb_b2)
    geo_bufs = (geo_b0, geo_b1, geo_b2)
    st_bufs = (st_b0, st_b1)
    sga = (sga0, sga1, sga2)
    sgb = (sgb0, sgb1, sgb2)
    so = (so0, so1)

    pltpu.sync_copy(idx_hbm.at[pl.ds(batch * P, P)], idx_v)

    def do_task(sel_hbm, out_hbm, prev_out):
        pltpu.sync_copy(sel_hbm.at[pl.ds(base, _RPW)], sel_v)

        @plsc.parallel_loop(0, _RPW // 32, 1, unroll=2)
        def group(g):
            s0 = sel_v[pl.ds(g * 32, 16)]
            s1 = sel_v[pl.ds(g * 32 + 16, 16)]
            l0 = jnp.zeros((16,), jnp.int32)
            l1 = jnp.zeros((16,), jnp.int32)
            ln = P
            while ln > 1:
                half = ln // 2
                v0 = plsc.load_gather(idx_v, [l0 + (half - 1)])
                v1 = plsc.load_gather(idx_v, [l1 + (half - 1)])
                l0 = l0 + (v0 < s0).astype(jnp.int32) * half
                l1 = l1 + (v1 < s1).astype(jnp.int32) * half
                ln -= half
            v0 = plsc.load_gather(idx_v, [l0])
            v1 = plsc.load_gather(idx_v, [l1])
            p0 = l0 + (v0 < s0).astype(jnp.int32)
            p1 = l1 + (v1 < s1).astype(jnp.int32)
            rgbi_v[pl.ds(g * 32, 16)] = s0 + batch * HW
            rgbi_v[pl.ds(g * 32 + 16, 16)] = s1 + batch * HW
            geoi_v[pl.ds(g * 32, 16)] = p0 + batch * P
            geoi_v[pl.ds(g * 32 + 16, 16)] = p1 + batch * P

        def fire(c):
            p = c % 3
            a = pltpu.async_copy(
                prgb.at[rgbi_v.at[pl.ds(c * _CH, _CH)]], rgb_bufs[p], sga[p])
            b = pltpu.async_copy(
                pgeo.at[geoi_v.at[pl.ds(c * _CH, _CH)]], geo_bufs[p], sgb[p])
            return a, b

        outcp = list(prev_out)
        pend = {0: fire(0), 1: fire(1)}
        for c in range(_NCH):
            p = c % 3
            q = c % 2
            if c + 2 < _NCH:
                pend[c + 2] = fire(c + 2)
            ca, cb = pend.pop(c)
            ca.wait()
            cb.wait()
            if outcp[q] is not None:
                outcp[q].wait()
                outcp[q] = None
            rb = rgb_bufs[p]
            gb = geo_bufs[p]
            st = st_bufs[q]
            himask = jnp.int32(-65536)

            @plsc.parallel_loop(0, _CH, 1, unroll=2)
            def _add(r):
                for j in range(FUS // 32):
                    s = pl.ds(j * 16, 16)
                    w_r = rb[r, s]
                    w_g = gb[r, s]
                    re = plsc.bitcast(w_r << 16, jnp.float32)
                    ge = plsc.bitcast(w_g << 16, jnp.float32)
                    ro = plsc.bitcast(w_r & himask, jnp.float32)
                    go = plsc.bitcast(w_g & himask, jnp.float32)
                    st[r, pl.ds(j * 16, 16)] = re + ge
                    st[r, pl.ds(FUS // 2 + j * 16, 16)] = ro + go
            outcp[q] = pltpu.async_copy(
                st, out_hbm.at[pl.ds(base + c * _CH, _CH)], so[q])
        return outcp

    pending = do_task(sel_m, out_m, [None, None])
    pending = do_task(sel_nm, out_nm, pending)
    for cp in pending:
        if cp is not None:
            cp.wait()


def _sc_side(prgb, pgeo, idx_hbm, sel_m, sel_nm):
    row = jax.ShapeDtypeStruct((B * M, FUS), jnp.float32)
    mesh = plsc.VectorSubcoreMesh(core_axis_name="c", subcore_axis_name="s")
    f = functools.partial(
        pl.kernel,
        mesh=mesh,
        out_type=[row, row],
        compiler_params=pltpu.CompilerParams(needs_layout_passes=False),
        scratch_types=[
            pltpu.VMEM((P,), jnp.int32),
            pltpu.VMEM((_RPW,), jnp.int32),
            pltpu.VMEM((_RPW,), jnp.int32),
            pltpu.VMEM((_RPW,), jnp.int32),
            pltpu.VMEM((_CH, FUS // 2), jnp.int32),
            pltpu.VMEM((_CH, FUS // 2), jnp.int32),
            pltpu.VMEM((_CH, FUS // 2), jnp.int32),
            pltpu.VMEM((_CH, FUS // 2), jnp.int32),
            pltpu.VMEM((_CH, FUS // 2), jnp.int32),
            pltpu.VMEM((_CH, FUS // 2), jnp.int32),
            pltpu.VMEM((_CH, FUS), jnp.float32),
            pltpu.VMEM((_CH, FUS), jnp.float32),
            pltpu.SemaphoreType.DMA,
            pltpu.SemaphoreType.DMA,
            pltpu.SemaphoreType.DMA,
            pltpu.SemaphoreType.DMA,
            pltpu.SemaphoreType.DMA,
            pltpu.SemaphoreType.DMA,
            pltpu.SemaphoreType.DMA,
            pltpu.SemaphoreType.DMA,
        ],
    )(_sc_side_body)
    return f(prgb, pgeo, idx_hbm, sel_m, sel_nm)


def kernel(soutput_f_l, soutput_f_r, rgb_f_l, rgb_f_r, W, b,
           idxs_l, idxs_r, matches, non_matches, start_idx, num_points):
    wt_rgb = W[:, :RGB].T
    wt_geo = W[:, RGB:].T
    bias = b.reshape(1, FUS)

    sel_lm = matches[:, :, 0].reshape(-1).astype(jnp.int32)
    sel_rm = matches[:, :, 1].reshape(-1).astype(jnp.int32)
    sel_lnm = non_matches[:, :, 0].reshape(-1).astype(jnp.int32)
    sel_rnm = non_matches[:, :, 1].reshape(-1).astype(jnp.int32)

    pgeo_l = _geo_proj(soutput_f_l, wt_geo, bias)
    prgb_l = _rgb_proj(rgb_f_l, wt_rgb)
    out_lm, out_lnm = _sc_side(prgb_l, pgeo_l, idxs_l.astype(jnp.int32),
                               sel_lm, sel_lnm)
    pgeo_r = _geo_proj(soutput_f_r, wt_geo, bias)
    prgb_r = _rgb_proj(rgb_f_r, wt_rgb)
    out_rm, out_rnm = _sc_side(prgb_r, pgeo_r, idxs_r.astype(jnp.int32),
                               sel_rm, sel_rnm)

    return (out_lm.reshape(B, M, FUS), out_lnm.reshape(B, M, FUS),
            out_rm.reshape(B, M, FUS), out_rnm.reshape(B, M, FUS))


# speedup vs baseline: 1.1240x; 1.1240x over previous
import functools

import jax
import jax.numpy as jnp
from jax import lax
from jax.experimental import pallas as pl
from jax.experimental.pallas import tpu as pltpu
from jax.experimental.pallas import tpu_sc as plsc

B = 4
P = 8192
HW = 9600
M = 4096
GEO = 64
RGB = 512
FUS = 256

_HW_TILE = 1920
_GEO_ROWS = 4096

_NC = 2
_NS = 16
_NW = _NC * _NS
_RPW = (B * M) // _NW
_CH = 64
_NCH = _RPW // _CH
_WPB = M // _RPW


def _round_bf16_bits(x):
    b = lax.bitcast_convert_type(x, jnp.uint32)
    return (b + jnp.uint32(0x7FFF) + ((b >> 16) & jnp.uint32(1))) >> 16


def _pack_halves(r):
    e = _round_bf16_bits(r[:, :FUS // 2])
    o = _round_bf16_bits(r[:, FUS // 2:])
    return lax.bitcast_convert_type(e | (o << 16), jnp.int32)


def _rgb_proj_body(rgb_ref, wt_ref, out_ref):
    out_ref[...] = _pack_halves(lax.dot_general(
        rgb_ref[0], wt_ref[...],
        dimension_numbers=(((0,), (0,)), ((), ())),
        preferred_element_type=jnp.float32))


def _rgb_proj(rgb_f, wt_rgb):
    nt = HW // _HW_TILE
    return pl.pallas_call(
        _rgb_proj_body,
        grid=(B, nt),
        in_specs=[
            pl.BlockSpec((1, RGB, _HW_TILE), lambda b, t: (b, 0, t)),
            pl.BlockSpec((RGB, FUS), lambda b, t: (0, 0)),
        ],
        out_specs=pl.BlockSpec((_HW_TILE, FUS // 2), lambda b, t: (b * nt + t, 0)),
        out_shape=jax.ShapeDtypeStruct((B * HW, FUS // 2), jnp.int32),
    )(rgb_f, wt_rgb)


def _geo_proj_body(geo_ref, wt_ref, b_ref, out_ref):
    out_ref[...] = _pack_halves(lax.dot_general(
        geo_ref[...], wt_ref[...],
        dimension_numbers=(((1,), (0,)), ((), ())),
        preferred_element_type=jnp.float32) + b_ref[...])


def _geo_proj(geo, wt_geo, bias):
    return pl.pallas_call(
        _geo_proj_body,
        grid=((B * P) // _GEO_ROWS,),
        in_specs=[
            pl.BlockSpec((_GEO_ROWS, GEO), lambda r: (r, 0)),
            pl.BlockSpec((GEO, FUS), lambda r: (0, 0)),
            pl.BlockSpec((1, FUS), lambda r: (0, 0)),
        ],
        out_specs=pl.BlockSpec((_GEO_ROWS, FUS // 2), lambda r: (r, 0)),
        out_shape=jax.ShapeDtypeStruct((B * P, FUS // 2), jnp.int32),
    )(geo, wt_geo, bias)


def _sc_side_body(prgb, pgeo, idx_hbm, sel_m, sel_nm, out_m, out_nm,
                  idx_v, sel_v, rgbi_v, geoi_v,
                  rgb_b0, rgb_b1, rgb_b2, geo_b0, geo_b1, geo_b2,
                  st_b0, st_b1,
                  sga0, sga1, sga2, sgb0, sgb1, sgb2, so0, so1):
    wid = lax.axis_index("s") * _NC + lax.axis_index("c")
    batch = wid // _WPB
    base = wid * _RPW
    rgb_bufs = (rgb_b0, rgb_b1, rgb_b2)
    geo_bufs = (geo_b0, geo_b1, geo_b2)
    st_bufs = (st_b0, st_b1)
    sga = (sga0, sga1, sga2)
    sgb = (sgb0, sgb1, sgb2)
    so = (so0, so1)

    pltpu.sync_copy(idx_hbm.at[pl.ds(batch * P, P)], idx_v)

    def do_task(sel_hbm, out_hbm, prev_out):
        pltpu.sync_copy(sel_hbm.at[pl.ds(base, _RPW)], sel_v)

        @plsc.parallel_loop(0, _RPW // 32, 1, unroll=2)
        def group(g):
            s0 = sel_v[pl.ds(g * 32, 16)]
            s1 = sel_v[pl.ds(g * 32 + 16, 16)]
            l0 = jnp.zeros((16,), jnp.int32)
            l1 = jnp.zeros((16,), jnp.int32)
            ln = P
            while ln > 1:
                half = ln // 2
                v0 = plsc.load_gather(idx_v, [l0 + (half - 1)])
                v1 = plsc.load_gather(idx_v, [l1 + (half - 1)])
                l0 = l0 + (v0 < s0).astype(jnp.int32) * half
                l1 = l1 + (v1 < s1).astype(jnp.int32) * half
                ln -= half
            v0 = plsc.load_gather(idx_v, [l0])
            v1 = plsc.load_gather(idx_v, [l1])
            p0 = l0 + (v0 < s0).astype(jnp.int32)
            p1 = l1 + (v1 < s1).astype(jnp.int32)
            rgbi_v[pl.ds(g * 32, 16)] = s0 + batch * HW
            rgbi_v[pl.ds(g * 32 + 16, 16)] = s1 + batch * HW
            geoi_v[pl.ds(g * 32, 16)] = p0 + batch * P
            geoi_v[pl.ds(g * 32 + 16, 16)] = p1 + batch * P

        def fire(c):
            p = c % 3
            a = pltpu.async_copy(
                prgb.at[rgbi_v.at[pl.ds(c * _CH, _CH)]], rgb_bufs[p], sga[p])
            b = pltpu.async_copy(
                pgeo.at[geoi_v.at[pl.ds(c * _CH, _CH)]], geo_bufs[p], sgb[p])
            return a, b

        outcp = list(prev_out)
        pend = {0: fire(0), 1: fire(1)}
        for c in range(_NCH):
            p = c % 3
            q = c % 2
            if c + 2 < _NCH:
                pend[c + 2] = fire(c + 2)
            ca, cb = pend.pop(c)
            ca.wait()
            cb.wait()
            if outcp[q] is not None:
                outcp[q].wait()
                outcp[q] = None
            rb = rgb_bufs[p]
            gb = geo_bufs[p]
            st = st_bufs[q]
            himask = jnp.int32(-65536)

            @plsc.parallel_loop(0, _CH, 1, unroll=2)
            def _add(r):
                for j in range(FUS // 32):
                    s = pl.ds(j * 16, 16)
                    w_r = rb[r, s]
                    w_g = gb[r, s]
                    re = plsc.bitcast(w_r << 16, jnp.float32)
                    ge = plsc.bitcast(w_g << 16, jnp.float32)
                    ro = plsc.bitcast(w_r & himask, jnp.float32)
                    go = plsc.bitcast(w_g & himask, jnp.float32)
                    st[r, pl.ds(j * 16, 16)] = re + ge
                    st[r, pl.ds(FUS // 2 + j * 16, 16)] = ro + go
            outcp[q] = pltpu.async_copy(
                st, out_hbm.at[pl.ds(base + c * _CH, _CH)], so[q])
        return outcp

    pending = do_task(sel_m, out_m, [None, None])
    pending = do_task(sel_nm, out_nm, pending)
    for cp in pending:
        if cp is not None:
            cp.wait()


def _sc_side(prgb, pgeo, idx_hbm, sel_m, sel_nm):
    row = jax.ShapeDtypeStruct((B * M, FUS), jnp.float32)
    mesh = plsc.VectorSubcoreMesh(core_axis_name="c", subcore_axis_name="s")
    f = functools.partial(
        pl.kernel,
        mesh=mesh,
        out_type=[row, row],
        compiler_params=pltpu.CompilerParams(needs_layout_passes=False),
        scratch_types=[
            pltpu.VMEM((P,), jnp.int32),
            pltpu.VMEM((_RPW,), jnp.int32),
            pltpu.VMEM((_RPW,), jnp.int32),
            pltpu.VMEM((_RPW,), jnp.int32),
            pltpu.VMEM((_CH, FUS // 2), jnp.int32),
            pltpu.VMEM((_CH, FUS // 2), jnp.int32),
            pltpu.VMEM((_CH, FUS // 2), jnp.int32),
            pltpu.VMEM((_CH, FUS // 2), jnp.int32),
            pltpu.VMEM((_CH, FUS // 2), jnp.int32),
            pltpu.VMEM((_CH, FUS // 2), jnp.int32),
            pltpu.VMEM((_CH, FUS), jnp.float32),
            pltpu.VMEM((_CH, FUS), jnp.float32),
            pltpu.SemaphoreType.DMA,
            pltpu.SemaphoreType.DMA,
            pltpu.SemaphoreType.DMA,
            pltpu.SemaphoreType.DMA,
            pltpu.SemaphoreType.DMA,
            pltpu.SemaphoreType.DMA,
            pltpu.SemaphoreType.DMA,
            pltpu.SemaphoreType.DMA,
        ],
    )(_sc_side_body)
    return f(prgb, pgeo, idx_hbm, sel_m, sel_nm)


def kernel(soutput_f_l, soutput_f_r, rgb_f_l, rgb_f_r, W, b,
           idxs_l, idxs_r, matches, non_matches, start_idx, num_points):
    wt_rgb = W[:, :RGB].T
    wt_geo = W[:, RGB:].T
    bias = b.reshape(1, FUS)

    sel_lm = matches[:, :, 0].reshape(-1).astype(jnp.int32)
    sel_rm = matches[:, :, 1].reshape(-1).astype(jnp.int32)
    sel_lnm = non_matches[:, :, 0].reshape(-1).astype(jnp.int32)
    sel_rnm = non_matches[:, :, 1].reshape(-1).astype(jnp.int32)

    pgeo_l = _geo_proj(soutput_f_l, wt_geo, bias)
    prgb_l = _rgb_proj(rgb_f_l, wt_rgb)
    out_lm, out_lnm = _sc_side(prgb_l, pgeo_l, idxs_l.astype(jnp.int32),
                               sel_lm, sel_lnm)
    pgeo_r = _geo_proj(soutput_f_r, wt_geo, bias)
    prgb_r = _rgb_proj(rgb_f_r, wt_rgb)
    out_rm, out_rnm = _sc_side(prgb_r, pgeo_r, idxs_r.astype(jnp.int32),
                               sel_rm, sel_rnm)

    return (out_lm.reshape(B, M, FUS), out_lnm.reshape(B, M, FUS),
            out_rm.reshape(B, M, FUS), out_rnm.reshape(B, M, FUS))
